# Initial kernel scaffold; baseline (speedup 1.0000x reference)
import jax
import jax.numpy as jnp
from jax.experimental import pallas as pl


def kernel(reid_x, st_x, edge_index, reid_W1, reid_b1, reid_W2, reid_b2, st_W1, st_b1, st_W2, st_b2, cat_W1, cat_b1, cat_W2, cat_b2):
    raise NotImplementedError("write your pallas kernel here")


# speedup vs baseline: 5.4752x; 5.4752x over previous
import functools

import jax
import jax.numpy as jnp
from jax import lax
from jax.experimental import pallas as pl
from jax.experimental.pallas import tpu as pltpu
from jax.experimental.pallas import tpu_sc as plsc

N = 10000
NP = 10240
E = 320000
NC = 2
NS = 16
DT = 64
CHUNK = 125
ZROWS = 32
ROWS_PER_SUB = NP // NS


def _seg(tables, src2d, dst2d):
  nt = len(tables)
  tpc = nt // NC
  cps = (E // NS) // CHUNK

  @functools.partial(
      pl.kernel,
      out_type=tuple(jax.ShapeDtypeStruct((NP, DT), jnp.float32)
                     for _ in range(nt)),
      mesh=plsc.VectorSubcoreMesh(core_axis_name="c", subcore_axis_name="s"),
      compiler_params=pltpu.CompilerParams(use_tc_tiling_on_sc=False),
      scratch_types=[
          pltpu.VMEM((cps, CHUNK), jnp.int32),
          pltpu.VMEM((cps, CHUNK), jnp.int32),
          pltpu.VMEM((CHUNK, DT), jnp.float32),
          pltpu.VMEM((CHUNK, DT), jnp.float32),
          pltpu.VMEM((ZROWS, DT), jnp.float32),
          pltpu.VMEM_SHARED((NP, DT), jnp.float32),
          pltpu.SemaphoreType.DMA,
          pltpu.SemaphoreType.DMA,
      ],
  )
  def k(*refs):
    x_hbms = refs[:nt]
    src_hbm, dst_hbm = refs[nt], refs[nt + 1]
    o_hbms = refs[nt + 2:2 * nt + 2]
    srcv, dstv, buf0, buf1, zbuf, acc, sem0, sem1 = refs[2 * nt + 2:]

    cid = lax.axis_index("c")
    sid = lax.axis_index("s")
    row0 = sid * ROWS_PER_SUB

    @pl.loop(0, ZROWS)
    def _(r):
      @pl.loop(0, DT, step=16)
      def _(c):
        zbuf[r, pl.ds(c, 16)] = jnp.zeros((16,), jnp.float32)

    pltpu.sync_copy(src_hbm.at[pl.ds(sid * cps, cps)], srcv)
    pltpu.sync_copy(dst_hbm.at[pl.ds(sid * cps, cps)], dstv)

    def run(x_hbm, o_hbm):
      @pl.loop(0, ROWS_PER_SUB // ZROWS)
      def _(i):
        pltpu.sync_copy(zbuf, acc.at[pl.ds(row0 + i * ZROWS, ZROWS)])

      plsc.subcore_barrier()

      pltpu.async_copy(x_hbm.at[srcv.at[0]], buf0, sem0)

      @pl.loop(0, cps, step=2)
      def _(j):
        pltpu.make_async_copy(x_hbm.at[srcv.at[j]], buf0, sem0).wait()
        pltpu.async_copy(x_hbm.at[srcv.at[j + 1]], buf1, sem1)
        pltpu.sync_copy(buf0, acc.at[dstv.at[j]], add=True)
        pltpu.make_async_copy(x_hbm.at[srcv.at[j + 1]], buf1, sem1).wait()

        @pl.when(j + 2 < cps)
        def _():
          pltpu.async_copy(x_hbm.at[srcv.at[j + 2]], buf0, sem0)

        pltpu.sync_copy(buf1, acc.at[dstv.at[j + 1]], add=True)

      plsc.subcore_barrier()
      pltpu.sync_copy(acc.at[pl.ds(row0, ROWS_PER_SUB)],
                      o_hbm.at[pl.ds(row0, ROWS_PER_SUB)])
      plsc.subcore_barrier()

    for c in range(NC):
      @pl.when(cid == c)
      def _(c=c):
        for t in range(tpc):
          run(x_hbms[c * tpc + t], o_hbms[c * tpc + t])

  return k(*tables, src2d, dst2d)


def _deg(dst2d):
  cps = (E // (NC * NS)) // CHUNK

  @functools.partial(
      pl.kernel,
      out_type=(jax.ShapeDtypeStruct((NP, 16), jnp.float32),
                jax.ShapeDtypeStruct((NP, 16), jnp.float32)),
      mesh=plsc.VectorSubcoreMesh(core_axis_name="c", subcore_axis_name="s"),
      compiler_params=pltpu.CompilerParams(use_tc_tiling_on_sc=False),
      scratch_types=[
          pltpu.VMEM((cps, CHUNK), jnp.int32),
          pltpu.VMEM((CHUNK, 16), jnp.float32),
          pltpu.VMEM((ZROWS, 16), jnp.float32),
          pltpu.VMEM_SHARED((NP, 16), jnp.float32),
      ],
  )
  def k(dst_hbm, o0_hbm, o1_hbm, dstv, ones_v, zbuf, acc):
    cid = lax.axis_index("c")
    sid = lax.axis_index("s")

    @pl.loop(0, ZROWS)
    def _(r):
      zbuf[r, pl.ds(0, 16)] = jnp.zeros((16,), jnp.float32)

    @pl.loop(0, CHUNK)
    def _(r):
      ones_v[r, pl.ds(0, 16)] = jnp.ones((16,), jnp.float32)

    row0 = sid * ROWS_PER_SUB

    @pl.loop(0, ROWS_PER_SUB // ZROWS)
    def _(i):
      pltpu.sync_copy(zbuf, acc.at[pl.ds(row0 + i * ZROWS, ZROWS)])

    chunk0 = (cid * NS + sid) * cps
    pltpu.sync_copy(dst_hbm.at[pl.ds(chunk0, cps)], dstv)
    plsc.subcore_barrier()

    @pl.loop(0, cps)
    def _(j):
      pltpu.sync_copy(ones_v, acc.at[dstv.at[j]], add=True)

    plsc.subcore_barrier()
    row_slice = pl.ds(row0, ROWS_PER_SUB)

    @pl.when(cid == 0)
    def _():
      pltpu.sync_copy(acc.at[row_slice], o0_hbm.at[row_slice])

    @pl.when(cid == 1)
    def _():
      pltpu.sync_copy(acc.at[row_slice], o1_hbm.at[row_slice])

  return k(dst2d)


def _post(ps, d0, d1, W, b, relu, widths):
  BN = 2000
  Dout = W.shape[1]
  Din = W.shape[0]
  np_ = len(ps)

  def body(*refs):
    p_refs = refs[:np_]
    d0_ref, d1_ref, w_ref, b_ref = refs[np_:np_ + 4]
    out_refs = refs[np_ + 4:]
    deg = d0_ref[:, 0:1] + d1_ref[:, 0:1]
    inv = 1.0 / jnp.maximum(deg, 1.0)
    h = jnp.concatenate([p[...] * inv for p in p_refs], axis=1)
    y = jnp.dot(h, w_ref[...], preferred_element_type=jnp.float32) + b_ref[...]
    if relu:
      y = jnp.maximum(y, 0.0)
    off = 0
    for r, w in zip(out_refs, widths):
      r[...] = y[:, off:off + w]
      off += w

  grid = (N // BN,)
  return pl.pallas_call(
      body,
      grid=grid,
      in_specs=[pl.BlockSpec((BN, DT), lambda i: (i, 0)) for _ in ps] + [
          pl.BlockSpec((BN, 16), lambda i: (i, 0)),
          pl.BlockSpec((BN, 16), lambda i: (i, 0)),
          pl.BlockSpec((Din, Dout), lambda i: (0, 0)),
          pl.BlockSpec((1, Dout), lambda i: (0, 0)),
      ],
      out_specs=[pl.BlockSpec((BN, w), lambda i: (i, 0)) for w in widths],
      out_shape=[jax.ShapeDtypeStruct((N, w), jnp.float32) for w in widths],
  )(*ps, d0, d1, W, b)


def _blockdiag(Wa, Wb):
  Da, Oa = Wa.shape
  Db, Ob = Wb.shape
  W = jnp.zeros((Da + Db, Oa + Ob), jnp.float32)
  W = W.at[:Da, :Oa].set(Wa)
  W = W.at[Da:, Oa:].set(Wb)
  return W


def kernel(reid_x, st_x, edge_index, reid_W1, reid_b1, reid_W2, reid_b2,
           st_W1, st_b1, st_W2, st_b2, cat_W1, cat_b1, cat_W2, cat_b2):
  src2d = edge_index[0].reshape(E // CHUNK, CHUNK)
  dst2d = edge_index[1].reshape(E // CHUNK, CHUNK)

  d0, d1 = _deg(dst2d)

  x_tables = (reid_x[:, :DT], reid_x[:, DT:], st_x[:, :DT], st_x[:, DT:])
  p1 = _seg(x_tables, src2d, dst2d)
  Wbd1 = _blockdiag(reid_W1, st_W1)
  bbd1 = jnp.concatenate([reid_b1, st_b1]).reshape(1, -1)
  h1 = _post(p1, d0, d1, Wbd1, bbd1, True, (DT,) * 4)

  p2 = _seg(tuple(h1), src2d, dst2d)
  Wbd2 = _blockdiag(reid_W2, st_W2)
  bbd2 = jnp.concatenate([reid_b2, st_b2]).reshape(1, -1)
  r2 = _post(p2, d0, d1, Wbd2, bbd2, False, (DT,) * 4)

  p3 = _seg(tuple(r2), src2d, dst2d)
  c1 = _post(p3, d0, d1, cat_W1, cat_b1.reshape(1, -1), True, (DT, DT))

  p4 = _seg(tuple(c1), src2d, dst2d)
  (out,) = _post(p4, d0, d1, cat_W2, cat_b2.reshape(1, -1), False, (128,))
  return out


# speedup vs baseline: 7.7750x; 1.4200x over previous
import functools

import jax
import jax.numpy as jnp
from jax import lax
from jax.experimental import pallas as pl
from jax.experimental.pallas import tpu as pltpu
from jax.experimental.pallas import tpu_sc as plsc

N = 10000
NP = 10240
E = 320000
NC = 2
NS = 16
DT = 64
CHUNK = 125
ZROWS = 32
ROWS_PER_SUB = NP // NS


def _seg(tables, src2d, dst2d):
  nt = len(tables)
  tpc = nt // NC
  cps = (E // NS) // CHUNK

  @functools.partial(
      pl.kernel,
      out_type=tuple(jax.ShapeDtypeStruct((NP, DT), jnp.float32)
                     for _ in range(nt)),
      mesh=plsc.VectorSubcoreMesh(core_axis_name="c", subcore_axis_name="s"),
      compiler_params=pltpu.CompilerParams(use_tc_tiling_on_sc=False),
      scratch_types=[
          pltpu.VMEM((cps, CHUNK), jnp.int32),
          pltpu.VMEM((cps, CHUNK), jnp.int32),
          pltpu.VMEM((CHUNK, DT), jnp.float32),
          pltpu.VMEM((CHUNK, DT), jnp.float32),
          pltpu.VMEM((CHUNK, DT), jnp.float32),
          pltpu.VMEM((CHUNK, DT), jnp.float32),
          pltpu.VMEM((ZROWS, DT), jnp.float32),
          pltpu.VMEM_SHARED((NP, DT), jnp.float32),
          pltpu.SemaphoreType.DMA,
          pltpu.SemaphoreType.DMA,
          pltpu.SemaphoreType.DMA,
          pltpu.SemaphoreType.DMA,
          pltpu.SemaphoreType.DMA,
          pltpu.SemaphoreType.DMA,
          pltpu.SemaphoreType.DMA,
          pltpu.SemaphoreType.DMA,
      ],
  )
  def k(*refs):
    x_hbms = refs[:nt]
    src_hbm, dst_hbm = refs[nt], refs[nt + 1]
    o_hbms = refs[nt + 2:2 * nt + 2]
    (srcv, dstv, buf0, buf1, buf2, buf3, zbuf, acc,
     g0, g1, g2, g3, s0, s1, s2, s3) = refs[2 * nt + 2:]
    bufs = (buf0, buf1, buf2, buf3)
    gsems = (g0, g1, g2, g3)
    ssems = (s0, s1, s2, s3)
    NB = 4

    cid = lax.axis_index("c")
    sid = lax.axis_index("s")
    row0 = sid * ROWS_PER_SUB

    @pl.loop(0, ZROWS)
    def _(r):
      @pl.loop(0, DT, step=16)
      def _(c):
        zbuf[r, pl.ds(c, 16)] = jnp.zeros((16,), jnp.float32)

    pltpu.sync_copy(src_hbm.at[pl.ds(sid * cps, cps)], srcv)
    pltpu.sync_copy(dst_hbm.at[pl.ds(sid * cps, cps)], dstv)

    def run(x_hbm, o_hbm):
      @pl.loop(0, ROWS_PER_SUB // ZROWS)
      def _(i):
        pltpu.sync_copy(zbuf, acc.at[pl.ds(row0 + i * ZROWS, ZROWS)])

      plsc.subcore_barrier()

      for b in range(NB):
        pltpu.async_copy(x_hbm.at[srcv.at[b]], bufs[b], gsems[b])

      @pl.loop(0, cps, step=NB)
      def _(j):
        for b in range(NB):
          pltpu.make_async_copy(x_hbm.at[srcv.at[j + b]], bufs[b],
                                gsems[b]).wait()
          pltpu.async_copy(bufs[b], acc.at[dstv.at[j + b]], ssems[b],
                           add=True)
        for b in range(NB):
          @pl.when(j + NB + b < cps)
          def _(b=b):
            pltpu.make_async_copy(bufs[b], acc.at[dstv.at[j + b]],
                                  ssems[b]).wait()
            pltpu.async_copy(x_hbm.at[srcv.at[j + NB + b]], bufs[b], gsems[b])

      for b in range(NB):
        pltpu.make_async_copy(bufs[b], acc.at[dstv.at[cps - NB + b]],
                              ssems[b]).wait()

      plsc.subcore_barrier()
      pltpu.sync_copy(acc.at[pl.ds(row0, ROWS_PER_SUB)],
                      o_hbm.at[pl.ds(row0, ROWS_PER_SUB)])
      plsc.subcore_barrier()

    for c in range(NC):
      @pl.when(cid == c)
      def _(c=c):
        for t in range(tpc):
          run(x_hbms[c * tpc + t], o_hbms[c * tpc + t])

  return k(*tables, src2d, dst2d)


def _deg(dst2d):
  cps = (E // (NC * NS)) // CHUNK

  @functools.partial(
      pl.kernel,
      out_type=(jax.ShapeDtypeStruct((NP, 16), jnp.float32),
                jax.ShapeDtypeStruct((NP, 16), jnp.float32)),
      mesh=plsc.VectorSubcoreMesh(core_axis_name="c", subcore_axis_name="s"),
      compiler_params=pltpu.CompilerParams(use_tc_tiling_on_sc=False),
      scratch_types=[
          pltpu.VMEM((cps, CHUNK), jnp.int32),
          pltpu.VMEM((CHUNK, 16), jnp.float32),
          pltpu.VMEM((ZROWS, 16), jnp.float32),
          pltpu.VMEM_SHARED((NP, 16), jnp.float32),
      ],
  )
  def k(dst_hbm, o0_hbm, o1_hbm, dstv, ones_v, zbuf, acc):
    cid = lax.axis_index("c")
    sid = lax.axis_index("s")

    @pl.loop(0, ZROWS)
    def _(r):
      zbuf[r, pl.ds(0, 16)] = jnp.zeros((16,), jnp.float32)

    @pl.loop(0, CHUNK)
    def _(r):
      ones_v[r, pl.ds(0, 16)] = jnp.ones((16,), jnp.float32)

    row0 = sid * ROWS_PER_SUB

    @pl.loop(0, ROWS_PER_SUB // ZROWS)
    def _(i):
      pltpu.sync_copy(zbuf, acc.at[pl.ds(row0 + i * ZROWS, ZROWS)])

    chunk0 = (cid * NS + sid) * cps
    pltpu.sync_copy(dst_hbm.at[pl.ds(chunk0, cps)], dstv)
    plsc.subcore_barrier()

    @pl.loop(0, cps)
    def _(j):
      pltpu.sync_copy(ones_v, acc.at[dstv.at[j]], add=True)

    plsc.subcore_barrier()
    row_slice = pl.ds(row0, ROWS_PER_SUB)

    @pl.when(cid == 0)
    def _():
      pltpu.sync_copy(acc.at[row_slice], o0_hbm.at[row_slice])

    @pl.when(cid == 1)
    def _():
      pltpu.sync_copy(acc.at[row_slice], o1_hbm.at[row_slice])

  return k(dst2d)


def _post(ps, d0, d1, W, b, relu, widths):
  BN = 2000
  Dout = W.shape[1]
  Din = W.shape[0]
  np_ = len(ps)

  def body(*refs):
    p_refs = refs[:np_]
    d0_ref, d1_ref, w_ref, b_ref = refs[np_:np_ + 4]
    out_refs = refs[np_ + 4:]
    deg = d0_ref[:, 0:1] + d1_ref[:, 0:1]
    inv = 1.0 / jnp.maximum(deg, 1.0)
    h = jnp.concatenate([p[...] * inv for p in p_refs], axis=1)
    y = jnp.dot(h, w_ref[...], preferred_element_type=jnp.float32) + b_ref[...]
    if relu:
      y = jnp.maximum(y, 0.0)
    off = 0
    for r, w in zip(out_refs, widths):
      r[...] = y[:, off:off + w]
      off += w

  grid = (N // BN,)
  return pl.pallas_call(
      body,
      grid=grid,
      in_specs=[pl.BlockSpec((BN, DT), lambda i: (i, 0)) for _ in ps] + [
          pl.BlockSpec((BN, 16), lambda i: (i, 0)),
          pl.BlockSpec((BN, 16), lambda i: (i, 0)),
          pl.BlockSpec((Din, Dout), lambda i: (0, 0)),
          pl.BlockSpec((1, Dout), lambda i: (0, 0)),
      ],
      out_specs=[pl.BlockSpec((BN, w), lambda i: (i, 0)) for w in widths],
      out_shape=[jax.ShapeDtypeStruct((N, w), jnp.float32) for w in widths],
  )(*ps, d0, d1, W, b)


def _blockdiag(Wa, Wb):
  Da, Oa = Wa.shape
  Db, Ob = Wb.shape
  W = jnp.zeros((Da + Db, Oa + Ob), jnp.float32)
  W = W.at[:Da, :Oa].set(Wa)
  W = W.at[Da:, Oa:].set(Wb)
  return W


def kernel(reid_x, st_x, edge_index, reid_W1, reid_b1, reid_W2, reid_b2,
           st_W1, st_b1, st_W2, st_b2, cat_W1, cat_b1, cat_W2, cat_b2):
  src2d = edge_index[0].reshape(E // CHUNK, CHUNK)
  dst2d = edge_index[1].reshape(E // CHUNK, CHUNK)

  d0, d1 = _deg(dst2d)

  x_tables = (reid_x[:, :DT], reid_x[:, DT:], st_x[:, :DT], st_x[:, DT:])
  p1 = _seg(x_tables, src2d, dst2d)
  Wbd1 = _blockdiag(reid_W1, st_W1)
  bbd1 = jnp.concatenate([reid_b1, st_b1]).reshape(1, -1)
  h1 = _post(p1, d0, d1, Wbd1, bbd1, True, (DT,) * 4)

  p2 = _seg(tuple(h1), src2d, dst2d)
  Wbd2 = _blockdiag(reid_W2, st_W2)
  bbd2 = jnp.concatenate([reid_b2, st_b2]).reshape(1, -1)
  r2 = _post(p2, d0, d1, Wbd2, bbd2, False, (DT,) * 4)

  p3 = _seg(tuple(r2), src2d, dst2d)
  c1 = _post(p3, d0, d1, cat_W1, cat_b1.reshape(1, -1), True, (DT, DT))

  p4 = _seg(tuple(c1), src2d, dst2d)
  (out,) = _post(p4, d0, d1, cat_W2, cat_b2.reshape(1, -1), False, (128,))
  return out


# speedup vs baseline: 7.9086x; 1.0172x over previous
import functools

import jax
import jax.numpy as jnp
from jax import lax
from jax.experimental import pallas as pl
from jax.experimental.pallas import tpu as pltpu
from jax.experimental.pallas import tpu_sc as plsc

N = 10000
NP = 10240
E = 320000
NC = 2
NS = 16
DT = 64
CHUNK = 125
ZROWS = 32
NB = 5
ROWS_PER_SUB = NP // NS


def _seg(tables, src2d, dst2d):
  nt = len(tables)
  tpc = nt // NC
  cps = (E // NS) // CHUNK

  @functools.partial(
      pl.kernel,
      out_type=tuple(jax.ShapeDtypeStruct((NP, DT), jnp.float32)
                     for _ in range(nt)),
      mesh=plsc.VectorSubcoreMesh(core_axis_name="c", subcore_axis_name="s"),
      compiler_params=pltpu.CompilerParams(use_tc_tiling_on_sc=False),
      scratch_types=[
          pltpu.VMEM((cps, CHUNK), jnp.int32),
          pltpu.VMEM((cps, CHUNK), jnp.int32),
      ] + [pltpu.VMEM((CHUNK, DT), jnp.float32)] * NB + [
          pltpu.VMEM((ZROWS, DT), jnp.float32),
          pltpu.VMEM_SHARED((NP, DT), jnp.float32),
      ] + [pltpu.SemaphoreType.DMA] * (2 * NB),
  )
  def k(*refs):
    x_hbms = refs[:nt]
    src_hbm, dst_hbm = refs[nt], refs[nt + 1]
    o_hbms = refs[nt + 2:2 * nt + 2]
    rest = refs[2 * nt + 2:]
    srcv, dstv = rest[0], rest[1]
    bufs = rest[2:2 + NB]
    zbuf, acc = rest[2 + NB], rest[3 + NB]
    gsems = rest[4 + NB:4 + 2 * NB]
    ssems = rest[4 + 2 * NB:4 + 3 * NB]

    cid = lax.axis_index("c")
    sid = lax.axis_index("s")
    row0 = sid * ROWS_PER_SUB

    @pl.loop(0, ZROWS)
    def _(r):
      @pl.loop(0, DT, step=16)
      def _(c):
        zbuf[r, pl.ds(c, 16)] = jnp.zeros((16,), jnp.float32)

    pltpu.sync_copy(src_hbm.at[pl.ds(sid * cps, cps)], srcv)
    pltpu.sync_copy(dst_hbm.at[pl.ds(sid * cps, cps)], dstv)

    def run(x_hbm, o_hbm):
      @pl.loop(0, ROWS_PER_SUB // ZROWS)
      def _(i):
        pltpu.sync_copy(zbuf, acc.at[pl.ds(row0 + i * ZROWS, ZROWS)])

      plsc.subcore_barrier()

      for b in range(NB):
        pltpu.async_copy(x_hbm.at[srcv.at[b]], bufs[b], gsems[b])

      @pl.loop(0, cps, step=NB)
      def _(j):
        for b in range(NB):
          pltpu.make_async_copy(x_hbm.at[srcv.at[j + b]], bufs[b],
                                gsems[b]).wait()
          pltpu.async_copy(bufs[b], acc.at[dstv.at[j + b]], ssems[b],
                           add=True)
        for b in range(NB):
          @pl.when(j + NB + b < cps)
          def _(b=b):
            pltpu.make_async_copy(bufs[b], acc.at[dstv.at[j + b]],
                                  ssems[b]).wait()
            pltpu.async_copy(x_hbm.at[srcv.at[j + NB + b]], bufs[b], gsems[b])

      for b in range(NB):
        pltpu.make_async_copy(bufs[b], acc.at[dstv.at[cps - NB + b]],
                              ssems[b]).wait()

      plsc.subcore_barrier()
      pltpu.sync_copy(acc.at[pl.ds(row0, ROWS_PER_SUB)],
                      o_hbm.at[pl.ds(row0, ROWS_PER_SUB)])
      plsc.subcore_barrier()

    for c in range(NC):
      @pl.when(cid == c)
      def _(c=c):
        for t in range(tpc):
          run(x_hbms[c * tpc + t], o_hbms[c * tpc + t])

  return k(*tables, src2d, dst2d)


def _deg(dst2d):
  cps = (E // (NC * NS)) // CHUNK

  @functools.partial(
      pl.kernel,
      out_type=(jax.ShapeDtypeStruct((NP, 16), jnp.float32),
                jax.ShapeDtypeStruct((NP, 16), jnp.float32)),
      mesh=plsc.VectorSubcoreMesh(core_axis_name="c", subcore_axis_name="s"),
      compiler_params=pltpu.CompilerParams(use_tc_tiling_on_sc=False),
      scratch_types=[
          pltpu.VMEM((cps, CHUNK), jnp.int32),
          pltpu.VMEM((CHUNK, 16), jnp.float32),
          pltpu.VMEM((ZROWS, 16), jnp.float32),
          pltpu.VMEM_SHARED((NP, 16), jnp.float32),
      ],
  )
  def k(dst_hbm, o0_hbm, o1_hbm, dstv, ones_v, zbuf, acc):
    cid = lax.axis_index("c")
    sid = lax.axis_index("s")

    @pl.loop(0, ZROWS)
    def _(r):
      zbuf[r, pl.ds(0, 16)] = jnp.zeros((16,), jnp.float32)

    @pl.loop(0, CHUNK)
    def _(r):
      ones_v[r, pl.ds(0, 16)] = jnp.ones((16,), jnp.float32)

    row0 = sid * ROWS_PER_SUB

    @pl.loop(0, ROWS_PER_SUB // ZROWS)
    def _(i):
      pltpu.sync_copy(zbuf, acc.at[pl.ds(row0 + i * ZROWS, ZROWS)])

    chunk0 = (cid * NS + sid) * cps
    pltpu.sync_copy(dst_hbm.at[pl.ds(chunk0, cps)], dstv)
    plsc.subcore_barrier()

    @pl.loop(0, cps)
    def _(j):
      pltpu.sync_copy(ones_v, acc.at[dstv.at[j]], add=True)

    plsc.subcore_barrier()
    row_slice = pl.ds(row0, ROWS_PER_SUB)

    @pl.when(cid == 0)
    def _():
      pltpu.sync_copy(acc.at[row_slice], o0_hbm.at[row_slice])

    @pl.when(cid == 1)
    def _():
      pltpu.sync_copy(acc.at[row_slice], o1_hbm.at[row_slice])

  return k(dst2d)


def _post(ps, d0, d1, W, b, relu, widths):
  BN = 2000
  Dout = W.shape[1]
  Din = W.shape[0]
  np_ = len(ps)

  def body(*refs):
    p_refs = refs[:np_]
    d0_ref, d1_ref, w_ref, b_ref = refs[np_:np_ + 4]
    out_refs = refs[np_ + 4:]
    deg = d0_ref[:, 0:1] + d1_ref[:, 0:1]
    inv = 1.0 / jnp.maximum(deg, 1.0)
    h = jnp.concatenate([p[...] * inv for p in p_refs], axis=1)
    y = jnp.dot(h, w_ref[...], preferred_element_type=jnp.float32) + b_ref[...]
    if relu:
      y = jnp.maximum(y, 0.0)
    off = 0
    for r, w in zip(out_refs, widths):
      r[...] = y[:, off:off + w]
      off += w

  grid = (N // BN,)
  return pl.pallas_call(
      body,
      grid=grid,
      in_specs=[pl.BlockSpec((BN, DT), lambda i: (i, 0)) for _ in ps] + [
          pl.BlockSpec((BN, 16), lambda i: (i, 0)),
          pl.BlockSpec((BN, 16), lambda i: (i, 0)),
          pl.BlockSpec((Din, Dout), lambda i: (0, 0)),
          pl.BlockSpec((1, Dout), lambda i: (0, 0)),
      ],
      out_specs=[pl.BlockSpec((BN, w), lambda i: (i, 0)) for w in widths],
      out_shape=[jax.ShapeDtypeStruct((N, w), jnp.float32) for w in widths],
  )(*ps, d0, d1, W, b)


def _blockdiag(Wa, Wb):
  Da, Oa = Wa.shape
  Db, Ob = Wb.shape
  W = jnp.zeros((Da + Db, Oa + Ob), jnp.float32)
  W = W.at[:Da, :Oa].set(Wa)
  W = W.at[Da:, Oa:].set(Wb)
  return W


def kernel(reid_x, st_x, edge_index, reid_W1, reid_b1, reid_W2, reid_b2,
           st_W1, st_b1, st_W2, st_b2, cat_W1, cat_b1, cat_W2, cat_b2):
  src2d = edge_index[0].reshape(E // CHUNK, CHUNK)
  dst2d = edge_index[1].reshape(E // CHUNK, CHUNK)

  d0, d1 = _deg(dst2d)

  x_tables = (reid_x[:, :DT], reid_x[:, DT:], st_x[:, :DT], st_x[:, DT:])
  p1 = _seg(x_tables, src2d, dst2d)
  Wbd1 = _blockdiag(reid_W1, st_W1)
  bbd1 = jnp.concatenate([reid_b1, st_b1]).reshape(1, -1)
  h1 = _post(p1, d0, d1, Wbd1, bbd1, True, (DT,) * 4)

  p2 = _seg(tuple(h1), src2d, dst2d)
  Wbd2 = _blockdiag(reid_W2, st_W2)
  bbd2 = jnp.concatenate([reid_b2, st_b2]).reshape(1, -1)
  r2 = _post(p2, d0, d1, Wbd2, bbd2, False, (DT,) * 4)

  p3 = _seg(tuple(r2), src2d, dst2d)
  c1 = _post(p3, d0, d1, cat_W1, cat_b1.reshape(1, -1), True, (DT, DT))

  p4 = _seg(tuple(c1), src2d, dst2d)
  (out,) = _post(p4, d0, d1, cat_W2, cat_b2.reshape(1, -1), False, (128,))
  return out
